# baseline (speedup 1.0000x reference)
import functools

import jax
import jax.numpy as jnp
from jax import lax
from jax.experimental import pallas as pl
from jax.experimental.pallas import tpu as pltpu
from jax.experimental.pallas import tpu_sc as plsc

NNODE = 10000
SIGMA = 0.5
VDF = 1.0

NC, NS, LANES = 2, 16, 16
NW = NC * NS
CHUNK = 128
NBUF = 3
N_PAD = 10000
ROWS_PER_TILE = N_PAD // NS
FLUSH_ROWS = 125

TN = 1000


def _mm_stats_body(x_ref, w_ref, b_ref, y_ref, st_ref, acc1, acc2):
    i = pl.program_id(0)
    y = jnp.dot(x_ref[...], w_ref[...], preferred_element_type=jnp.float32)
    y = y + b_ref[...]
    y_ref[...] = y

    @pl.when(i == 0)
    def _():
        acc1[...] = jnp.zeros_like(acc1)
        acc2[...] = jnp.zeros_like(acc2)

    acc1[...] += jnp.sum(y, axis=0, keepdims=True)
    acc2[...] += jnp.sum(y * y, axis=0, keepdims=True)

    @pl.when(i == pl.num_programs(0) - 1)
    def _():
        st_ref[...] = jnp.concatenate([acc1[...], acc2[...]], axis=0)


def _mm_stats(x, w, b):
    n, k = x.shape
    d = w.shape[1]
    y, st = pl.pallas_call(
        _mm_stats_body,
        grid=(n // TN,),
        in_specs=[
            pl.BlockSpec((TN, k), lambda i: (i, 0)),
            pl.BlockSpec((k, d), lambda i: (0, 0)),
            pl.BlockSpec((1, d), lambda i: (0, 0)),
        ],
        out_specs=[
            pl.BlockSpec((TN, d), lambda i: (i, 0)),
            pl.BlockSpec((2, d), lambda i: (0, 0)),
        ],
        out_shape=[
            jax.ShapeDtypeStruct((n, d), jnp.float32),
            jax.ShapeDtypeStruct((2, d), jnp.float32),
        ],
        scratch_shapes=[
            pltpu.VMEM((1, d), jnp.float32),
            pltpu.VMEM((1, d), jnp.float32),
        ],
    )(x, w, b.reshape(1, d))
    return y, st


def _bn_act_body(y_ref, st_ref, g_ref, bb_ref, o_ref, *, relu, n):
    st = st_ref[...]
    mean = st[0:1, :] / n
    var = st[1:2, :] / n - mean * mean
    scale = g_ref[...] * lax.rsqrt(var + 1e-5)
    shift = bb_ref[...] - mean * scale
    o = y_ref[...] * scale + shift
    if relu:
        o = jnp.maximum(o, 0.0)
    o_ref[...] = o


def _bn_act(y, st, g, bb, relu):
    n, d = y.shape
    return pl.pallas_call(
        functools.partial(_bn_act_body, relu=relu, n=float(n)),
        grid=(n // TN,),
        in_specs=[
            pl.BlockSpec((TN, d), lambda i: (i, 0)),
            pl.BlockSpec((2, d), lambda i: (0, 0)),
            pl.BlockSpec((1, d), lambda i: (0, 0)),
            pl.BlockSpec((1, d), lambda i: (0, 0)),
        ],
        out_specs=pl.BlockSpec((TN, d), lambda i: (i, 0)),
        out_shape=jax.ShapeDtypeStruct((n, d), jnp.float32),
    )(y, st, g.reshape(1, d), bb.reshape(1, d))


def _mm_chunks_body(x_ref, w_ref, *o_refs, out_w):
    xv = x_ref[...]
    col = 0
    for o_ref, wo in zip(o_refs, out_w):
        o_ref[...] = jnp.dot(xv, w_ref[:, col:col + wo],
                             preferred_element_type=jnp.float32)
        col += wo


def _mm_chunks(x, w, out_w):
    n, k = x.shape
    d = w.shape[1]
    outs = pl.pallas_call(
        functools.partial(_mm_chunks_body, out_w=tuple(out_w)),
        grid=(n // TN,),
        in_specs=[
            pl.BlockSpec((TN, k), lambda i: (i, 0)),
            pl.BlockSpec((k, d), lambda i: (0, 0)),
        ],
        out_specs=[pl.BlockSpec((TN, wo), lambda i: (i, 0)) for wo in out_w],
        out_shape=[jax.ShapeDtypeStruct((n, wo), jnp.float32) for wo in out_w],
    )(x, w)
    return list(outs)


def _gnn_mix_body(*refs, n_in, in_w, out_w, sigma):
    p_refs = refs[:n_in]
    a_ref = refs[n_in]
    g_ref = refs[n_in + 1]
    o_refs = refs[n_in + 2:]
    mixes = []
    col = 0
    for j in range(n_in):
        pj = p_refs[j][...]
        h = jnp.maximum(pj[0] + pj[1], 0.0)
        mixes.append((1.0 - sigma) * h + sigma * a_ref[:, col:col + in_w[j]])
        col += in_w[j]
    col_out = 0
    for c, o_ref in enumerate(o_refs):
        wo = out_w[c]
        acc = None
        col_in = 0
        for j in range(n_in):
            t = jnp.dot(mixes[j],
                        g_ref[col_in:col_in + in_w[j], col_out:col_out + wo],
                        preferred_element_type=jnp.float32)
            acc = t if acc is None else acc + t
            col_in += in_w[j]
        o_ref[...] = acc
        col_out += wo


def _gnn_mix(p_list, a, g, out_w):
    n = a.shape[0]
    in_w = [p.shape[2] for p in p_list]
    d = g.shape[1]
    outs = pl.pallas_call(
        functools.partial(_gnn_mix_body, n_in=len(p_list), in_w=tuple(in_w),
                          out_w=tuple(out_w), sigma=SIGMA),
        grid=(n // TN,),
        in_specs=(
            [pl.BlockSpec((2, TN, wj), lambda i: (0, i, 0)) for wj in in_w]
            + [pl.BlockSpec((TN, a.shape[1]), lambda i: (i, 0)),
               pl.BlockSpec(g.shape, lambda i: (0, 0))]
        ),
        out_specs=[pl.BlockSpec((TN, wo), lambda i: (i, 0)) for wo in out_w],
        out_shape=[jax.ShapeDtypeStruct((n, wo), jnp.float32) for wo in out_w],
    )(*p_list, a, g)
    return list(outs)


def _softmax_body(p_ref, o_ref):
    p = p_ref[...]
    h = p[0] + p[1]
    colid = lax.broadcasted_iota(jnp.int32, h.shape, 1)
    valid = colid < 10
    hm = jnp.where(valid, h, -jnp.inf)
    m = jnp.max(hm, axis=1, keepdims=True)
    e = jnp.where(valid, jnp.exp(h - m), 0.0)
    o_ref[...] = (e / jnp.sum(e, axis=1, keepdims=True))[:, :10]


def _softmax10(p):
    n = NNODE
    return pl.pallas_call(
        _softmax_body,
        grid=(n // TN,),
        in_specs=[pl.BlockSpec((2, TN, 16), lambda i: (0, i, 0))],
        out_specs=pl.BlockSpec((TN, 10), lambda i: (i, 0)),
        out_shape=jax.ShapeDtypeStruct((n, 10), jnp.float32),
    )(p)


def _q_body(z_ref, ct_ref, o_ref):
    z = z_ref[...]
    ct = ct_ref[...]
    zz = jnp.sum(z * z, axis=1, keepdims=True)
    cc = jnp.sum(ct * ct, axis=0, keepdims=True)
    d2 = zz - 2.0 * jnp.dot(z, ct, preferred_element_type=jnp.float32) + cc
    num = 1.0 / (1.0 + d2 / VDF)
    o_ref[...] = num / jnp.sum(num, axis=1, keepdims=True)


def _q_assign(z3, cluster_t):
    n = z3.shape[0]
    return pl.pallas_call(
        _q_body,
        grid=(n // TN,),
        in_specs=[
            pl.BlockSpec((TN, 16), lambda i: (i, 0)),
            pl.BlockSpec((16, 10), lambda i: (0, 0)),
        ],
        out_specs=pl.BlockSpec((TN, 10), lambda i: (i, 0)),
        out_shape=jax.ShapeDtypeStruct((n, 10), jnp.float32),
    )(z3, cluster_t)


def _heads_body(a_ref, wx_ref, bx_ref, wm_ref, bm_ref, wd_ref, bd_ref,
                wp_ref, bp_ref, xb_ref, mean_ref, disp_ref, pi_ref):
    a = a_ref[...]
    xb_ref[...] = jnp.dot(a, wx_ref[...],
                          preferred_element_type=jnp.float32) + bx_ref[...]
    ym = jnp.dot(a, wm_ref[...], preferred_element_type=jnp.float32) + bm_ref[...]
    mean_ref[...] = jnp.clip(jnp.exp(ym), 1e-5, 1e6)
    yd = jnp.dot(a, wd_ref[...], preferred_element_type=jnp.float32) + bd_ref[...]
    sp = jnp.maximum(yd, 0.0) + jnp.log1p(jnp.exp(-jnp.abs(yd)))
    disp_ref[...] = jnp.clip(sp, 1e-4, 1e4)
    yp = jnp.dot(a, wp_ref[...], preferred_element_type=jnp.float32) + bp_ref[...]
    pi_ref[...] = 1.0 / (1.0 + jnp.exp(-yp))


def _heads(a, wx, bx, wm, bm, wd, bd, wp_, bp):
    n, k = a.shape
    d = wx.shape[1]
    mat = lambda: pl.BlockSpec((k, d), lambda i: (0, 0))
    vec = lambda: pl.BlockSpec((1, d), lambda i: (0, 0))
    out = lambda: pl.BlockSpec((TN, d), lambda i: (i, 0))
    outs = pl.pallas_call(
        _heads_body,
        grid=(n // TN,),
        in_specs=[pl.BlockSpec((TN, k), lambda i: (i, 0)),
                  mat(), vec(), mat(), vec(), mat(), vec(), mat(), vec()],
        out_specs=[out() for _ in range(4)],
        out_shape=[jax.ShapeDtypeStruct((n, d), jnp.float32) for _ in range(4)],
    )(a, wx, bx.reshape(1, d), wm, bm.reshape(1, d),
      wd, bd.reshape(1, d), wp_, bp.reshape(1, d))
    return outs


def _spmm_body(m_hbm, src_hbm, dst_hbm, w_hbm, out_hbm,
               idx0, idx1, idx2, dstb0, dstb1, dstb2, wb0, wb1, wb2,
               rows0, rows1, rows2, shared,
               is0, is1, is2, ds0, ds1, ds2, ws0, ws1, ws2,
               rs0, rs1, rs2, ss0, ss1, ss2,
               *, width, nch_tile):
    buf = rows0
    c = lax.axis_index("c")
    s = lax.axis_index("s")
    wid = s * NC + c
    nq = width // LANES
    slot = (
        dict(idx=idx0, dstb=dstb0, wb=wb0, rows=rows0,
             isem=is0, dsem=ds0, wsem=ws0, rsem=rs0, ssem=ss0),
        dict(idx=idx1, dstb=dstb1, wb=wb1, rows=rows1,
             isem=is1, dsem=ds1, wsem=ws1, rsem=rs1, ssem=ss1),
        dict(idx=idx2, dstb=dstb2, wb=wb2, rows=rows2,
             isem=is2, dsem=ds2, wsem=ws2, rsem=rs2, ssem=ss2),
    )

    def zrow(i, _):
        for q in range(nq):
            buf[i, pl.ds(q * LANES, LANES)] = jnp.zeros((LANES,), jnp.float32)
        return 0
    lax.fori_loop(0, FLUSH_ROWS, zrow, 0)
    for r in range(ROWS_PER_TILE // FLUSH_ROWS):
        pltpu.sync_copy(
            buf.at[pl.ds(0, FLUSH_ROWS)],
            shared.at[pl.ds(s * ROWS_PER_TILE + r * FLUSH_ROWS, FLUSH_ROWS)])

    base = wid * nch_tile

    def iwload(b, j):
        sl = slot[b]
        pltpu.async_copy(src_hbm.at[j], sl['idx'], sl['isem'])
        pltpu.async_copy(w_hbm.at[j], sl['wb'], sl['wsem'])

    def iwwait(b):
        sl = slot[b]
        pltpu.make_async_copy(src_hbm.at[0], sl['idx'], sl['isem']).wait()
        pltpu.make_async_copy(w_hbm.at[0], sl['wb'], sl['wsem']).wait()

    def dload(b, j):
        sl = slot[b]
        pltpu.async_copy(dst_hbm.at[j], sl['dstb'], sl['dsem'])

    def dwait(b):
        sl = slot[b]
        pltpu.make_async_copy(dst_hbm.at[0], sl['dstb'], sl['dsem']).wait()

    def gstart(b):
        sl = slot[b]
        pltpu.async_copy(m_hbm.at[sl['idx']], sl['rows'], sl['rsem'])

    def gwait(b):
        sl = slot[b]
        pltpu.make_async_copy(m_hbm.at[sl['idx']], sl['rows'], sl['rsem']).wait()

    def sstart(b):
        sl = slot[b]
        pltpu.async_copy(sl['rows'], shared.at[sl['dstb']], sl['ssem'], add=True)

    def swait(b):
        sl = slot[b]
        pltpu.make_async_copy(sl['rows'], shared.at[sl['dstb']], sl['ssem']).wait()

    for b in range(NBUF):
        iwload(b, base + b)
        dload(b, base + b)
    iwwait(0)
    gstart(0)
    plsc.subcore_barrier()

    def grp(g, _):
        for b in range(NBUF):
            i = g * NBUF + b
            sl = slot[b]
            nb = (b + 1) % NBUF

            @pl.when(i + 1 < nch_tile)
            def _():
                @pl.when(i >= 2)
                def _():
                    swait(nb)
                    dload(nb, base + i + 1)
                iwwait(nb)
                gstart(nb)

            gwait(b)
            rows, wb = sl['rows'], sl['wb']

            def group_body(gg, _):
                w16 = wb[pl.ds(gg * LANES, LANES)]
                for kk in range(LANES):
                    wk = w16[kk]
                    k = gg * LANES + kk
                    for q in range(nq):
                        qs = pl.ds(q * LANES, LANES)
                        rows[k, qs] = rows[k, qs] * wk
                return 0
            lax.fori_loop(0, CHUNK // LANES, group_body, 0)

            dwait(b)
            sstart(b)

            @pl.when(i + 3 < nch_tile)
            def _():
                iwload(b, base + i + 3)
        return 0
    lax.fori_loop(0, nch_tile // NBUF, grp, 0)
    for b in range(NBUF):
        swait(b)
    plsc.subcore_barrier()

    for r in range(ROWS_PER_TILE // FLUSH_ROWS):
        off = s * ROWS_PER_TILE + r * FLUSH_ROWS
        pltpu.sync_copy(shared.at[pl.ds(off, FLUSH_ROWS)],
                        buf.at[pl.ds(0, FLUSH_ROWS)])
        pltpu.sync_copy(buf.at[pl.ds(0, FLUSH_ROWS)],
                        out_hbm.at[c, pl.ds(off, FLUSH_ROWS)])


def _spmm_chunk(m, srcp, dstp, wp):
    width = m.shape[1]
    nch = dstp.shape[0]
    nch_tile = nch // NW
    mesh = plsc.VectorSubcoreMesh(core_axis_name="c", subcore_axis_name="s",
                                  num_cores=NC, num_subcores=NS)
    f = pl.kernel(
        functools.partial(_spmm_body, width=width, nch_tile=nch_tile),
        out_type=jax.ShapeDtypeStruct((NC, N_PAD, width), jnp.float32),
        mesh=mesh,
        compiler_params=pltpu.CompilerParams(use_tc_tiling_on_sc=False),
        scratch_types=(
            [pltpu.VMEM((CHUNK,), jnp.int32)] * 6
            + [pltpu.VMEM((CHUNK,), jnp.float32)] * 3
            + [pltpu.VMEM((CHUNK, width), jnp.float32)] * 3
            + [pltpu.VMEM_SHARED((N_PAD, width), jnp.float32)]
            + [pltpu.SemaphoreType.DMA] * 15
        ),
    )
    return f(m, srcp, dstp, wp)


def kernel(x, edge_index, edge_weight, params):
    p = params

    y1, st1 = _mm_stats(x, p['e1']['W'], p['e1']['b'])
    a1 = _bn_act(y1, st1, p['bn1']['g'], p['bn1']['b'], relu=True)
    y2, st2 = _mm_stats(a1, p['e2']['W'], p['e2']['b'])
    a2 = _bn_act(y2, st2, p['bn2']['g'], p['bn2']['b'], relu=True)
    y3, st3 = _mm_stats(a2, p['e3']['W'], p['e3']['b'])
    a3 = _bn_act(y3, st3, p['bn3']['g'], p['bn3']['b'], relu=True)
    yz1, stz1 = _mm_stats(a3, p['z1']['W'], p['z1']['b'])
    az1 = _bn_act(yz1, stz1, p['bn4']['g'], p['bn4']['b'], relu=False)
    yz2, stz2 = _mm_stats(az1, p['z2']['W'], p['z2']['b'])
    az2 = _bn_act(yz2, stz2, p['bn5']['g'], p['bn5']['b'], relu=False)
    yz3, stz3 = _mm_stats(az2, p['z3']['W'], p['z3']['b'])
    z3 = _bn_act(yz3, stz3, p['bn6']['g'], p['bn6']['b'], relu=False)
    yd1, std1 = _mm_stats(z3, p['d1']['W'], p['d1']['b'])
    ad1 = _bn_act(yd1, std1, p['bn7']['g'], p['bn7']['b'], relu=True)
    yd2, std2 = _mm_stats(ad1, p['d2']['W'], p['d2']['b'])
    ad2 = _bn_act(yd2, std2, p['bn8']['g'], p['bn8']['b'], relu=True)
    yd3, std3 = _mm_stats(ad2, p['d3']['W'], p['d3']['b'])
    ad3 = _bn_act(yd3, std3, p['bn9']['g'], p['bn9']['b'], relu=True)

    x_bar, mean_, disp_, pi_ = _heads(
        ad3, p['xbar']['W'], p['xbar']['b'], p['dm']['W'], p['dm']['b'],
        p['dd']['W'], p['dd']['b'], p['dp']['W'], p['dp']['b'])

    q = _q_assign(z3, p['cluster'].T)

    src = edge_index[0]
    dst = edge_index[1]
    e = src.shape[0]
    nch = -(-e // (CHUNK * NW * NBUF)) * (NW * NBUF)
    pad = nch * CHUNK - e
    srcp = jnp.concatenate([src, jnp.zeros((pad,), jnp.int32)]).reshape(nch, CHUNK)
    wp = jnp.concatenate([edge_weight, jnp.zeros((pad,), jnp.float32)]).reshape(nch, CHUNK)
    dstp = jnp.concatenate([dst, jnp.zeros((pad,), jnp.int32)]).reshape(nch, CHUNK)

    def spmm_all(m_chunks):
        return [_spmm_chunk(m, srcp, dstp, wp) for m in m_chunks]

    m = _mm_chunks(x, p['g1'], [128, 128])
    ps = spmm_all(m)
    m = _gnn_mix(ps, a1, p['g2'], [128, 128])
    ps = spmm_all(m)
    m = _gnn_mix(ps, a2, p['g3'], [128, 128, 128, 128])
    ps = spmm_all(m)
    m = _gnn_mix(ps, a3, p['g4'], [64])
    ps = spmm_all(m)
    m = _gnn_mix(ps, az1, p['g5'], [32])
    ps = spmm_all(m)
    m = _gnn_mix(ps, az2, p['g6'], [16])
    ps = spmm_all(m)
    g7p = jnp.pad(p['g7'], ((0, 0), (0, 6)))
    m = _gnn_mix(ps, z3, g7p, [16])
    ps = spmm_all(m)
    predict = _softmax10(ps[0])

    return (x_bar, q, predict, z3, mean_, disp_, pi_)


# speedup vs baseline: 2.3495x; 2.3495x over previous
import functools

import jax
import jax.numpy as jnp
from jax import lax
from jax.experimental import pallas as pl
from jax.experimental.pallas import tpu as pltpu
from jax.experimental.pallas import tpu_sc as plsc

NNODE = 10000
SIGMA = 0.5
VDF = 1.0

NC, NS, LANES = 2, 16, 16
NW = NC * NS
CHUNK = 96
NBUF = 3
N_PAD = 10000
ROWS_PER_TILE = N_PAD // NS
FLUSH_ROWS = 125

TN = 1000


def _mm_stats_body(x_ref, w_ref, b_ref, y_ref, st_ref, acc1, acc2):
    i = pl.program_id(0)
    y = jnp.dot(x_ref[...], w_ref[...], preferred_element_type=jnp.float32)
    y = y + b_ref[...]
    y_ref[...] = y

    @pl.when(i == 0)
    def _():
        acc1[...] = jnp.zeros_like(acc1)
        acc2[...] = jnp.zeros_like(acc2)

    acc1[...] += jnp.sum(y, axis=0, keepdims=True)
    acc2[...] += jnp.sum(y * y, axis=0, keepdims=True)

    @pl.when(i == pl.num_programs(0) - 1)
    def _():
        st_ref[...] = jnp.concatenate([acc1[...], acc2[...]], axis=0)


def _mm_stats(x, w, b):
    n, k = x.shape
    d = w.shape[1]
    y, st = pl.pallas_call(
        _mm_stats_body,
        grid=(n // TN,),
        in_specs=[
            pl.BlockSpec((TN, k), lambda i: (i, 0)),
            pl.BlockSpec((k, d), lambda i: (0, 0)),
            pl.BlockSpec((1, d), lambda i: (0, 0)),
        ],
        out_specs=[
            pl.BlockSpec((TN, d), lambda i: (i, 0)),
            pl.BlockSpec((2, d), lambda i: (0, 0)),
        ],
        out_shape=[
            jax.ShapeDtypeStruct((n, d), jnp.float32),
            jax.ShapeDtypeStruct((2, d), jnp.float32),
        ],
        scratch_shapes=[
            pltpu.VMEM((1, d), jnp.float32),
            pltpu.VMEM((1, d), jnp.float32),
        ],
    )(x, w, b.reshape(1, d))
    return y, st


def _bn_act_body(y_ref, st_ref, g_ref, bb_ref, o_ref, *, relu, n):
    st = st_ref[...]
    mean = st[0:1, :] / n
    var = st[1:2, :] / n - mean * mean
    scale = g_ref[...] * lax.rsqrt(var + 1e-5)
    shift = bb_ref[...] - mean * scale
    o = y_ref[...] * scale + shift
    if relu:
        o = jnp.maximum(o, 0.0)
    o_ref[...] = o


def _bn_act(y, st, g, bb, relu):
    n, d = y.shape
    return pl.pallas_call(
        functools.partial(_bn_act_body, relu=relu, n=float(n)),
        grid=(n // TN,),
        in_specs=[
            pl.BlockSpec((TN, d), lambda i: (i, 0)),
            pl.BlockSpec((2, d), lambda i: (0, 0)),
            pl.BlockSpec((1, d), lambda i: (0, 0)),
            pl.BlockSpec((1, d), lambda i: (0, 0)),
        ],
        out_specs=pl.BlockSpec((TN, d), lambda i: (i, 0)),
        out_shape=jax.ShapeDtypeStruct((n, d), jnp.float32),
    )(y, st, g.reshape(1, d), bb.reshape(1, d))


def _mm_chunks_body(x_ref, w_ref, *o_refs, out_w):
    xv = x_ref[...]
    col = 0
    for o_ref, wo in zip(o_refs, out_w):
        o_ref[...] = jnp.dot(xv, w_ref[:, col:col + wo],
                             preferred_element_type=jnp.float32)
        col += wo


def _mm_chunks(x, w, out_w):
    n, k = x.shape
    d = w.shape[1]
    outs = pl.pallas_call(
        functools.partial(_mm_chunks_body, out_w=tuple(out_w)),
        grid=(n // TN,),
        in_specs=[
            pl.BlockSpec((TN, k), lambda i: (i, 0)),
            pl.BlockSpec((k, d), lambda i: (0, 0)),
        ],
        out_specs=[pl.BlockSpec((TN, wo), lambda i: (i, 0)) for wo in out_w],
        out_shape=[jax.ShapeDtypeStruct((n, wo), jnp.float32) for wo in out_w],
    )(x, w)
    return list(outs)


def _gnn_mix_body(*refs, n_in, in_w, out_w, sigma):
    p_refs = refs[:n_in]
    a_ref = refs[n_in]
    g_ref = refs[n_in + 1]
    o_refs = refs[n_in + 2:]
    mixes = []
    col = 0
    for j in range(n_in):
        pj = p_refs[j][...]
        h = jnp.maximum(pj[0] + pj[1], 0.0)
        mixes.append((1.0 - sigma) * h + sigma * a_ref[:, col:col + in_w[j]])
        col += in_w[j]
    col_out = 0
    for c, o_ref in enumerate(o_refs):
        wo = out_w[c]
        acc = None
        col_in = 0
        for j in range(n_in):
            t = jnp.dot(mixes[j],
                        g_ref[col_in:col_in + in_w[j], col_out:col_out + wo],
                        preferred_element_type=jnp.float32)
            acc = t if acc is None else acc + t
            col_in += in_w[j]
        o_ref[...] = acc
        col_out += wo


def _gnn_mix(p_list, a, g, out_w):
    n = a.shape[0]
    in_w = [p.shape[2] for p in p_list]
    d = g.shape[1]
    outs = pl.pallas_call(
        functools.partial(_gnn_mix_body, n_in=len(p_list), in_w=tuple(in_w),
                          out_w=tuple(out_w), sigma=SIGMA),
        grid=(n // TN,),
        in_specs=(
            [pl.BlockSpec((2, TN, wj), lambda i: (0, i, 0)) for wj in in_w]
            + [pl.BlockSpec((TN, a.shape[1]), lambda i: (i, 0)),
               pl.BlockSpec(g.shape, lambda i: (0, 0))]
        ),
        out_specs=[pl.BlockSpec((TN, wo), lambda i: (i, 0)) for wo in out_w],
        out_shape=[jax.ShapeDtypeStruct((n, wo), jnp.float32) for wo in out_w],
    )(*p_list, a, g)
    return list(outs)


def _softmax_body(p_ref, o_ref):
    p = p_ref[...]
    h = p[0] + p[1]
    colid = lax.broadcasted_iota(jnp.int32, h.shape, 1)
    valid = colid < 10
    hm = jnp.where(valid, h, -jnp.inf)
    m = jnp.max(hm, axis=1, keepdims=True)
    e = jnp.where(valid, jnp.exp(h - m), 0.0)
    o_ref[...] = (e / jnp.sum(e, axis=1, keepdims=True))[:, :10]


def _softmax10(p):
    n = NNODE
    return pl.pallas_call(
        _softmax_body,
        grid=(n // TN,),
        in_specs=[pl.BlockSpec((2, TN, 16), lambda i: (0, i, 0))],
        out_specs=pl.BlockSpec((TN, 10), lambda i: (i, 0)),
        out_shape=jax.ShapeDtypeStruct((n, 10), jnp.float32),
    )(p)


def _q_body(z_ref, ct_ref, o_ref):
    z = z_ref[...]
    ct = ct_ref[...]
    zz = jnp.sum(z * z, axis=1, keepdims=True)
    cc = jnp.sum(ct * ct, axis=0, keepdims=True)
    d2 = zz - 2.0 * jnp.dot(z, ct, preferred_element_type=jnp.float32) + cc
    num = 1.0 / (1.0 + d2 / VDF)
    o_ref[...] = num / jnp.sum(num, axis=1, keepdims=True)


def _q_assign(z3, cluster_t):
    n = z3.shape[0]
    return pl.pallas_call(
        _q_body,
        grid=(n // TN,),
        in_specs=[
            pl.BlockSpec((TN, 16), lambda i: (i, 0)),
            pl.BlockSpec((16, 10), lambda i: (0, 0)),
        ],
        out_specs=pl.BlockSpec((TN, 10), lambda i: (i, 0)),
        out_shape=jax.ShapeDtypeStruct((n, 10), jnp.float32),
    )(z3, cluster_t)


def _heads_body(a_ref, wx_ref, bx_ref, wm_ref, bm_ref, wd_ref, bd_ref,
                wp_ref, bp_ref, xb_ref, mean_ref, disp_ref, pi_ref):
    a = a_ref[...]
    xb_ref[...] = jnp.dot(a, wx_ref[...],
                          preferred_element_type=jnp.float32) + bx_ref[...]
    ym = jnp.dot(a, wm_ref[...], preferred_element_type=jnp.float32) + bm_ref[...]
    mean_ref[...] = jnp.clip(jnp.exp(ym), 1e-5, 1e6)
    yd = jnp.dot(a, wd_ref[...], preferred_element_type=jnp.float32) + bd_ref[...]
    sp = jnp.maximum(yd, 0.0) + jnp.log1p(jnp.exp(-jnp.abs(yd)))
    disp_ref[...] = jnp.clip(sp, 1e-4, 1e4)
    yp = jnp.dot(a, wp_ref[...], preferred_element_type=jnp.float32) + bp_ref[...]
    pi_ref[...] = 1.0 / (1.0 + jnp.exp(-yp))


def _heads(a, wx, bx, wm, bm, wd, bd, wp_, bp):
    n, k = a.shape
    d = wx.shape[1]
    mat = lambda: pl.BlockSpec((k, d), lambda i: (0, 0))
    vec = lambda: pl.BlockSpec((1, d), lambda i: (0, 0))
    out = lambda: pl.BlockSpec((TN, d), lambda i: (i, 0))
    outs = pl.pallas_call(
        _heads_body,
        grid=(n // TN,),
        in_specs=[pl.BlockSpec((TN, k), lambda i: (i, 0)),
                  mat(), vec(), mat(), vec(), mat(), vec(), mat(), vec()],
        out_specs=[out() for _ in range(4)],
        out_shape=[jax.ShapeDtypeStruct((n, d), jnp.float32) for _ in range(4)],
    )(a, wx, bx.reshape(1, d), wm, bm.reshape(1, d),
      wd, bd.reshape(1, d), wp_, bp.reshape(1, d))
    return outs


def _spmm_body(m_hbm, src_hbm, dst_hbm, w_hbm, out_hbm,
               idx0, idx1, idx2, dstb0, dstb1, dstb2, wb0, wb1, wb2,
               rows0, rows1, rows2, shared,
               is0, is1, is2, ds0, ds1, ds2, ws0, ws1, ws2,
               rs0, rs1, rs2, ss0, ss1, ss2,
               *, width, nch_tile):
    buf = rows0
    c = lax.axis_index("c")
    s = lax.axis_index("s")
    wid = s * NC + c
    nq = width // LANES
    slot = (
        dict(idx=idx0, dstb=dstb0, wb=wb0, rows=rows0,
             isem=is0, dsem=ds0, wsem=ws0, rsem=rs0, ssem=ss0),
        dict(idx=idx1, dstb=dstb1, wb=wb1, rows=rows1,
             isem=is1, dsem=ds1, wsem=ws1, rsem=rs1, ssem=ss1),
        dict(idx=idx2, dstb=dstb2, wb=wb2, rows=rows2,
             isem=is2, dsem=ds2, wsem=ws2, rsem=rs2, ssem=ss2),
    )

    def zrow(i, _):
        for q in range(nq):
            buf[i, pl.ds(q * LANES, LANES)] = jnp.zeros((LANES,), jnp.float32)
        return 0
    lax.fori_loop(0, FLUSH_ROWS, zrow, 0)
    for r in range(ROWS_PER_TILE // FLUSH_ROWS):
        pltpu.sync_copy(
            buf.at[pl.ds(0, FLUSH_ROWS)],
            shared.at[pl.ds(s * ROWS_PER_TILE + r * FLUSH_ROWS, FLUSH_ROWS)])

    base = wid * nch_tile

    def iwload(b, j):
        sl = slot[b]
        pltpu.async_copy(src_hbm.at[j], sl['idx'], sl['isem'])
        pltpu.async_copy(w_hbm.at[j], sl['wb'], sl['wsem'])

    def iwwait(b):
        sl = slot[b]
        pltpu.make_async_copy(src_hbm.at[0], sl['idx'], sl['isem']).wait()
        pltpu.make_async_copy(w_hbm.at[0], sl['wb'], sl['wsem']).wait()

    def dload(b, j):
        sl = slot[b]
        pltpu.async_copy(dst_hbm.at[j], sl['dstb'], sl['dsem'])

    def dwait(b):
        sl = slot[b]
        pltpu.make_async_copy(dst_hbm.at[0], sl['dstb'], sl['dsem']).wait()

    def gstart(b):
        sl = slot[b]
        pltpu.async_copy(m_hbm.at[sl['idx']], sl['rows'], sl['rsem'])

    def gwait(b):
        sl = slot[b]
        pltpu.make_async_copy(m_hbm.at[sl['idx']], sl['rows'], sl['rsem']).wait()

    def sstart(b):
        sl = slot[b]
        pltpu.async_copy(sl['rows'], shared.at[sl['dstb']], sl['ssem'], add=True)

    def swait(b):
        sl = slot[b]
        pltpu.make_async_copy(sl['rows'], shared.at[sl['dstb']], sl['ssem']).wait()

    for b in range(NBUF):
        iwload(b, base + b)
        dload(b, base + b)
    iwwait(0)
    gstart(0)
    plsc.subcore_barrier()

    def grp(g, _):
        for b in range(NBUF):
            i = g * NBUF + b
            sl = slot[b]
            nb = (b + 1) % NBUF

            @pl.when(i + 1 < nch_tile)
            def _():
                @pl.when(i >= 2)
                def _():
                    swait(nb)
                    dload(nb, base + i + 1)
                iwwait(nb)
                gstart(nb)

            gwait(b)
            rows, wb = sl['rows'], sl['wb']

            def group_body(gg, _):
                w16 = wb[pl.ds(gg * LANES, LANES)]
                for kk in range(LANES):
                    wk = w16[kk]
                    k = gg * LANES + kk
                    for q in range(nq):
                        qs = pl.ds(q * LANES, LANES)
                        rows[k, qs] = rows[k, qs] * wk
                return 0
            lax.fori_loop(0, CHUNK // LANES, group_body, 0)

            dwait(b)
            sstart(b)

            @pl.when(i + 3 < nch_tile)
            def _():
                iwload(b, base + i + 3)
        return 0
    lax.fori_loop(0, nch_tile // NBUF, grp, 0)
    for b in range(NBUF):
        swait(b)
    plsc.subcore_barrier()

    for r in range(ROWS_PER_TILE // FLUSH_ROWS):
        off = s * ROWS_PER_TILE + r * FLUSH_ROWS
        pltpu.sync_copy(shared.at[pl.ds(off, FLUSH_ROWS)],
                        buf.at[pl.ds(0, FLUSH_ROWS)])
        pltpu.sync_copy(buf.at[pl.ds(0, FLUSH_ROWS)],
                        out_hbm.at[c, pl.ds(off, FLUSH_ROWS)])


def _spmm_chunk(m, srcp, dstp, wp):
    width = m.shape[1]
    nch = dstp.shape[0]
    nch_tile = nch // NW
    mesh = plsc.VectorSubcoreMesh(core_axis_name="c", subcore_axis_name="s",
                                  num_cores=NC, num_subcores=NS)
    f = pl.kernel(
        functools.partial(_spmm_body, width=width, nch_tile=nch_tile),
        out_type=jax.ShapeDtypeStruct((NC, N_PAD, width), jnp.float32),
        mesh=mesh,
        compiler_params=pltpu.CompilerParams(use_tc_tiling_on_sc=False),
        scratch_types=(
            [pltpu.VMEM((CHUNK,), jnp.int32)] * 6
            + [pltpu.VMEM((CHUNK,), jnp.float32)] * 3
            + [pltpu.VMEM((CHUNK, width), jnp.float32)] * 3
            + [pltpu.VMEM_SHARED((N_PAD, width), jnp.float32)]
            + [pltpu.SemaphoreType.DMA] * 15
        ),
    )
    return f(m, srcp, dstp, wp)


def kernel(x, edge_index, edge_weight, params):
    p = params

    y1, st1 = _mm_stats(x, p['e1']['W'], p['e1']['b'])
    a1 = _bn_act(y1, st1, p['bn1']['g'], p['bn1']['b'], relu=True)
    y2, st2 = _mm_stats(a1, p['e2']['W'], p['e2']['b'])
    a2 = _bn_act(y2, st2, p['bn2']['g'], p['bn2']['b'], relu=True)
    y3, st3 = _mm_stats(a2, p['e3']['W'], p['e3']['b'])
    a3 = _bn_act(y3, st3, p['bn3']['g'], p['bn3']['b'], relu=True)
    yz1, stz1 = _mm_stats(a3, p['z1']['W'], p['z1']['b'])
    az1 = _bn_act(yz1, stz1, p['bn4']['g'], p['bn4']['b'], relu=False)
    yz2, stz2 = _mm_stats(az1, p['z2']['W'], p['z2']['b'])
    az2 = _bn_act(yz2, stz2, p['bn5']['g'], p['bn5']['b'], relu=False)
    yz3, stz3 = _mm_stats(az2, p['z3']['W'], p['z3']['b'])
    z3 = _bn_act(yz3, stz3, p['bn6']['g'], p['bn6']['b'], relu=False)
    yd1, std1 = _mm_stats(z3, p['d1']['W'], p['d1']['b'])
    ad1 = _bn_act(yd1, std1, p['bn7']['g'], p['bn7']['b'], relu=True)
    yd2, std2 = _mm_stats(ad1, p['d2']['W'], p['d2']['b'])
    ad2 = _bn_act(yd2, std2, p['bn8']['g'], p['bn8']['b'], relu=True)
    yd3, std3 = _mm_stats(ad2, p['d3']['W'], p['d3']['b'])
    ad3 = _bn_act(yd3, std3, p['bn9']['g'], p['bn9']['b'], relu=True)

    x_bar, mean_, disp_, pi_ = _heads(
        ad3, p['xbar']['W'], p['xbar']['b'], p['dm']['W'], p['dm']['b'],
        p['dd']['W'], p['dd']['b'], p['dp']['W'], p['dp']['b'])

    q = _q_assign(z3, p['cluster'].T)

    src = edge_index[0]
    dst = edge_index[1]
    e = src.shape[0]
    nch = -(-e // (CHUNK * NW * NBUF)) * (NW * NBUF)
    pad = nch * CHUNK - e
    srcp = jnp.concatenate([src, jnp.zeros((pad,), jnp.int32)]).reshape(nch, CHUNK)
    wp = jnp.concatenate([edge_weight, jnp.zeros((pad,), jnp.float32)]).reshape(nch, CHUNK)
    dstp = jnp.concatenate([dst, jnp.zeros((pad,), jnp.int32)]).reshape(nch, CHUNK)

    def spmm_all(m_chunks):
        return [_spmm_chunk(m, srcp, dstp, wp) for m in m_chunks]

    m = _mm_chunks(x, p['g1'], [128, 128])
    ps = spmm_all(m)
    m = _gnn_mix(ps, a1, p['g2'], [128, 128])
    ps = spmm_all(m)
    m = _gnn_mix(ps, a2, p['g3'], [128, 128, 128, 128])
    ps = spmm_all(m)
    m = _gnn_mix(ps, a3, p['g4'], [64])
    ps = spmm_all(m)
    m = _gnn_mix(ps, az1, p['g5'], [32])
    ps = spmm_all(m)
    m = _gnn_mix(ps, az2, p['g6'], [16])
    ps = spmm_all(m)
    g7p = jnp.pad(p['g7'], ((0, 0), (0, 6)))
    m = _gnn_mix(ps, z3, g7p, [16])
    ps = spmm_all(m)
    predict = _softmax10(ps[0])

    return (x_bar, q, predict, z3, mean_, disp_, pi_)


# speedup vs baseline: 2.3544x; 1.0021x over previous
import functools

import jax
import jax.numpy as jnp
from jax import lax
from jax.experimental import pallas as pl
from jax.experimental.pallas import tpu as pltpu
from jax.experimental.pallas import tpu_sc as plsc

NNODE = 10000
SIGMA = 0.5
VDF = 1.0

NC, NS, LANES = 2, 16, 16
NW = NC * NS
CHUNK = 96
NBUF = 3
N_PAD = 10000
ROWS_PER_TILE = N_PAD // NS
FLUSH_ROWS = 125

TN = 1000


def _mm_stats_body(x_ref, w_ref, b_ref, y_ref, st_ref, acc1, acc2):
    i = pl.program_id(0)
    y = jnp.dot(x_ref[...], w_ref[...], preferred_element_type=jnp.float32)
    y = y + b_ref[...]
    y_ref[...] = y

    @pl.when(i == 0)
    def _():
        acc1[...] = jnp.zeros_like(acc1)
        acc2[...] = jnp.zeros_like(acc2)

    acc1[...] += jnp.sum(y, axis=0, keepdims=True)
    acc2[...] += jnp.sum(y * y, axis=0, keepdims=True)

    @pl.when(i == pl.num_programs(0) - 1)
    def _():
        st_ref[...] = jnp.concatenate([acc1[...], acc2[...]], axis=0)


def _mm_stats(x, w, b):
    n, k = x.shape
    d = w.shape[1]
    y, st = pl.pallas_call(
        _mm_stats_body,
        grid=(n // TN,),
        in_specs=[
            pl.BlockSpec((TN, k), lambda i: (i, 0)),
            pl.BlockSpec((k, d), lambda i: (0, 0)),
            pl.BlockSpec((1, d), lambda i: (0, 0)),
        ],
        out_specs=[
            pl.BlockSpec((TN, d), lambda i: (i, 0)),
            pl.BlockSpec((2, d), lambda i: (0, 0)),
        ],
        out_shape=[
            jax.ShapeDtypeStruct((n, d), jnp.float32),
            jax.ShapeDtypeStruct((2, d), jnp.float32),
        ],
        scratch_shapes=[
            pltpu.VMEM((1, d), jnp.float32),
            pltpu.VMEM((1, d), jnp.float32),
        ],
    )(x, w, b.reshape(1, d))
    return y, st


def _bn_act_body(y_ref, st_ref, g_ref, bb_ref, o_ref, *, relu, n):
    st = st_ref[...]
    mean = st[0:1, :] / n
    var = st[1:2, :] / n - mean * mean
    scale = g_ref[...] * lax.rsqrt(var + 1e-5)
    shift = bb_ref[...] - mean * scale
    o = y_ref[...] * scale + shift
    if relu:
        o = jnp.maximum(o, 0.0)
    o_ref[...] = o


def _bn_act(y, st, g, bb, relu):
    n, d = y.shape
    return pl.pallas_call(
        functools.partial(_bn_act_body, relu=relu, n=float(n)),
        grid=(n // TN,),
        in_specs=[
            pl.BlockSpec((TN, d), lambda i: (i, 0)),
            pl.BlockSpec((2, d), lambda i: (0, 0)),
            pl.BlockSpec((1, d), lambda i: (0, 0)),
            pl.BlockSpec((1, d), lambda i: (0, 0)),
        ],
        out_specs=pl.BlockSpec((TN, d), lambda i: (i, 0)),
        out_shape=jax.ShapeDtypeStruct((n, d), jnp.float32),
    )(y, st, g.reshape(1, d), bb.reshape(1, d))


def _mm_chunks_body(x_ref, w_ref, *o_refs, out_w):
    xv = x_ref[...]
    col = 0
    for o_ref, wo in zip(o_refs, out_w):
        o_ref[...] = jnp.dot(xv, w_ref[:, col:col + wo],
                             preferred_element_type=jnp.float32)
        col += wo


def _mm_chunks(x, w, out_w):
    n, k = x.shape
    d = w.shape[1]
    outs = pl.pallas_call(
        functools.partial(_mm_chunks_body, out_w=tuple(out_w)),
        grid=(n // TN,),
        in_specs=[
            pl.BlockSpec((TN, k), lambda i: (i, 0)),
            pl.BlockSpec((k, d), lambda i: (0, 0)),
        ],
        out_specs=[pl.BlockSpec((TN, wo), lambda i: (i, 0)) for wo in out_w],
        out_shape=[jax.ShapeDtypeStruct((n, wo), jnp.float32) for wo in out_w],
    )(x, w)
    return list(outs)


def _gnn_mix_body(*refs, n_in, in_w, out_w, sigma):
    p_refs = refs[:n_in]
    a_ref = refs[n_in]
    g_ref = refs[n_in + 1]
    o_refs = refs[n_in + 2:]
    mixes = []
    col = 0
    for j in range(n_in):
        pj = p_refs[j][...]
        h = jnp.maximum(pj[0] + pj[1], 0.0)
        mixes.append((1.0 - sigma) * h + sigma * a_ref[:, col:col + in_w[j]])
        col += in_w[j]
    col_out = 0
    for c, o_ref in enumerate(o_refs):
        wo = out_w[c]
        acc = None
        col_in = 0
        for j in range(n_in):
            t = jnp.dot(mixes[j],
                        g_ref[col_in:col_in + in_w[j], col_out:col_out + wo],
                        preferred_element_type=jnp.float32)
            acc = t if acc is None else acc + t
            col_in += in_w[j]
        o_ref[...] = acc
        col_out += wo


def _gnn_mix(p_list, a, g, out_w):
    n = a.shape[0]
    in_w = [p.shape[2] for p in p_list]
    d = g.shape[1]
    outs = pl.pallas_call(
        functools.partial(_gnn_mix_body, n_in=len(p_list), in_w=tuple(in_w),
                          out_w=tuple(out_w), sigma=SIGMA),
        grid=(n // TN,),
        in_specs=(
            [pl.BlockSpec((2, TN, wj), lambda i: (0, i, 0)) for wj in in_w]
            + [pl.BlockSpec((TN, a.shape[1]), lambda i: (i, 0)),
               pl.BlockSpec(g.shape, lambda i: (0, 0))]
        ),
        out_specs=[pl.BlockSpec((TN, wo), lambda i: (i, 0)) for wo in out_w],
        out_shape=[jax.ShapeDtypeStruct((n, wo), jnp.float32) for wo in out_w],
    )(*p_list, a, g)
    return list(outs)


def _softmax_body(p_ref, o_ref):
    p = p_ref[...]
    h = p[0] + p[1]
    colid = lax.broadcasted_iota(jnp.int32, h.shape, 1)
    valid = colid < 10
    hm = jnp.where(valid, h, -jnp.inf)
    m = jnp.max(hm, axis=1, keepdims=True)
    e = jnp.where(valid, jnp.exp(h - m), 0.0)
    o_ref[...] = (e / jnp.sum(e, axis=1, keepdims=True))[:, :10]


def _softmax10(p):
    n = NNODE
    return pl.pallas_call(
        _softmax_body,
        grid=(n // TN,),
        in_specs=[pl.BlockSpec((2, TN, 16), lambda i: (0, i, 0))],
        out_specs=pl.BlockSpec((TN, 10), lambda i: (i, 0)),
        out_shape=jax.ShapeDtypeStruct((n, 10), jnp.float32),
    )(p)


def _q_body(z_ref, ct_ref, o_ref):
    z = z_ref[...]
    ct = ct_ref[...]
    zz = jnp.sum(z * z, axis=1, keepdims=True)
    cc = jnp.sum(ct * ct, axis=0, keepdims=True)
    d2 = zz - 2.0 * jnp.dot(z, ct, preferred_element_type=jnp.float32) + cc
    num = 1.0 / (1.0 + d2 / VDF)
    o_ref[...] = num / jnp.sum(num, axis=1, keepdims=True)


def _q_assign(z3, cluster_t):
    n = z3.shape[0]
    return pl.pallas_call(
        _q_body,
        grid=(n // TN,),
        in_specs=[
            pl.BlockSpec((TN, 16), lambda i: (i, 0)),
            pl.BlockSpec((16, 10), lambda i: (0, 0)),
        ],
        out_specs=pl.BlockSpec((TN, 10), lambda i: (i, 0)),
        out_shape=jax.ShapeDtypeStruct((n, 10), jnp.float32),
    )(z3, cluster_t)


def _heads_body(a_ref, wx_ref, bx_ref, wm_ref, bm_ref, wd_ref, bd_ref,
                wp_ref, bp_ref, xb_ref, mean_ref, disp_ref, pi_ref):
    a = a_ref[...]
    xb_ref[...] = jnp.dot(a, wx_ref[...],
                          preferred_element_type=jnp.float32) + bx_ref[...]
    ym = jnp.dot(a, wm_ref[...], preferred_element_type=jnp.float32) + bm_ref[...]
    mean_ref[...] = jnp.clip(jnp.exp(ym), 1e-5, 1e6)
    yd = jnp.dot(a, wd_ref[...], preferred_element_type=jnp.float32) + bd_ref[...]
    sp = jnp.maximum(yd, 0.0) + jnp.log1p(jnp.exp(-jnp.abs(yd)))
    disp_ref[...] = jnp.clip(sp, 1e-4, 1e4)
    yp = jnp.dot(a, wp_ref[...], preferred_element_type=jnp.float32) + bp_ref[...]
    pi_ref[...] = 1.0 / (1.0 + jnp.exp(-yp))


def _heads(a, wx, bx, wm, bm, wd, bd, wp_, bp):
    n, k = a.shape
    d = wx.shape[1]
    mat = lambda: pl.BlockSpec((k, d), lambda i: (0, 0))
    vec = lambda: pl.BlockSpec((1, d), lambda i: (0, 0))
    out = lambda: pl.BlockSpec((TN, d), lambda i: (i, 0))
    outs = pl.pallas_call(
        _heads_body,
        grid=(n // TN,),
        in_specs=[pl.BlockSpec((TN, k), lambda i: (i, 0)),
                  mat(), vec(), mat(), vec(), mat(), vec(), mat(), vec()],
        out_specs=[out() for _ in range(4)],
        out_shape=[jax.ShapeDtypeStruct((n, d), jnp.float32) for _ in range(4)],
    )(a, wx, bx.reshape(1, d), wm, bm.reshape(1, d),
      wd, bd.reshape(1, d), wp_, bp.reshape(1, d))
    return outs


def _spmm_body(m_hbm, src_hbm, dst_hbm, w_hbm, out_hbm,
               idx0, idx1, idx2, dstb0, dstb1, dstb2, wb0, wb1, wb2,
               rows0, rows1, rows2, shared,
               is0, is1, is2, ds0, ds1, ds2, ws0, ws1, ws2,
               rs0, rs1, rs2, ss0, ss1, ss2,
               *, width, nch_tile):
    buf = rows0
    c = lax.axis_index("c")
    s = lax.axis_index("s")
    wid = s * NC + c
    nq = width // LANES
    slot = (
        dict(idx=idx0, dstb=dstb0, wb=wb0, rows=rows0,
             isem=is0, dsem=ds0, wsem=ws0, rsem=rs0, ssem=ss0),
        dict(idx=idx1, dstb=dstb1, wb=wb1, rows=rows1,
             isem=is1, dsem=ds1, wsem=ws1, rsem=rs1, ssem=ss1),
        dict(idx=idx2, dstb=dstb2, wb=wb2, rows=rows2,
             isem=is2, dsem=ds2, wsem=ws2, rsem=rs2, ssem=ss2),
    )

    def zrow(i, _):
        for q in range(nq):
            buf[i, pl.ds(q * LANES, LANES)] = jnp.zeros((LANES,), jnp.float32)
        return 0
    lax.fori_loop(0, FLUSH_ROWS, zrow, 0)
    for r in range(ROWS_PER_TILE // FLUSH_ROWS):
        pltpu.sync_copy(
            buf.at[pl.ds(0, FLUSH_ROWS)],
            shared.at[pl.ds(s * ROWS_PER_TILE + r * FLUSH_ROWS, FLUSH_ROWS)])

    base = wid * nch_tile

    def iwload(b, j):
        sl = slot[b]
        pltpu.async_copy(src_hbm.at[j], sl['idx'], sl['isem'])
        pltpu.async_copy(w_hbm.at[j], sl['wb'], sl['wsem'])

    def iwwait(b):
        sl = slot[b]
        pltpu.make_async_copy(src_hbm.at[0], sl['idx'], sl['isem']).wait()
        pltpu.make_async_copy(w_hbm.at[0], sl['wb'], sl['wsem']).wait()

    def dload(b, j):
        sl = slot[b]
        pltpu.async_copy(dst_hbm.at[j], sl['dstb'], sl['dsem'])

    def dwait(b):
        sl = slot[b]
        pltpu.make_async_copy(dst_hbm.at[0], sl['dstb'], sl['dsem']).wait()

    def gstart(b):
        sl = slot[b]
        pltpu.async_copy(m_hbm.at[sl['idx']], sl['rows'], sl['rsem'])

    def gwait(b):
        sl = slot[b]
        pltpu.make_async_copy(m_hbm.at[sl['idx']], sl['rows'], sl['rsem']).wait()

    def sstart(b):
        sl = slot[b]
        pltpu.async_copy(sl['rows'], shared.at[sl['dstb']], sl['ssem'], add=True)

    def swait(b):
        sl = slot[b]
        pltpu.make_async_copy(sl['rows'], shared.at[sl['dstb']], sl['ssem']).wait()

    for b in range(NBUF):
        iwload(b, base + b)
        dload(b, base + b)
    iwwait(0)
    gstart(0)
    plsc.subcore_barrier()

    def grp(g, _):
        for b in range(NBUF):
            i = g * NBUF + b
            sl = slot[b]
            nb = (b + 1) % NBUF

            @pl.when(i + 1 < nch_tile)
            def _():
                @pl.when(i >= 2)
                def _():
                    swait(nb)
                    dload(nb, base + i + 1)
                iwwait(nb)
                gstart(nb)

            gwait(b)
            rows, wb = sl['rows'], sl['wb']

            def group_body(gg, _):
                w16 = wb[pl.ds(gg * LANES, LANES)]
                for kk in range(LANES):
                    wk = w16[kk]
                    k = gg * LANES + kk
                    for q in range(nq):
                        qs = pl.ds(q * LANES, LANES)
                        rows[k, qs] = rows[k, qs] * wk
                return 0
            lax.fori_loop(0, CHUNK // LANES, group_body, 0)

            dwait(b)
            sstart(b)

            @pl.when(i + 3 < nch_tile)
            def _():
                iwload(b, base + i + 3)
        return 0
    lax.fori_loop(0, nch_tile // NBUF, grp, 0)
    for b in range(NBUF):
        swait(b)
    plsc.subcore_barrier()

    off = s * ROWS_PER_TILE
    pltpu.sync_copy(shared.at[pl.ds(off, ROWS_PER_TILE)],
                    out_hbm.at[c, pl.ds(off, ROWS_PER_TILE)])


def _spmm_chunk(m, srcp, dstp, wp):
    width = m.shape[1]
    nch = dstp.shape[0]
    nch_tile = nch // NW
    mesh = plsc.VectorSubcoreMesh(core_axis_name="c", subcore_axis_name="s",
                                  num_cores=NC, num_subcores=NS)
    f = pl.kernel(
        functools.partial(_spmm_body, width=width, nch_tile=nch_tile),
        out_type=jax.ShapeDtypeStruct((NC, N_PAD, width), jnp.float32),
        mesh=mesh,
        compiler_params=pltpu.CompilerParams(use_tc_tiling_on_sc=False),
        scratch_types=(
            [pltpu.VMEM((CHUNK,), jnp.int32)] * 6
            + [pltpu.VMEM((CHUNK,), jnp.float32)] * 3
            + [pltpu.VMEM((CHUNK, width), jnp.float32)] * 3
            + [pltpu.VMEM_SHARED((N_PAD, width), jnp.float32)]
            + [pltpu.SemaphoreType.DMA] * 15
        ),
    )
    return f(m, srcp, dstp, wp)


def kernel(x, edge_index, edge_weight, params):
    p = params

    y1, st1 = _mm_stats(x, p['e1']['W'], p['e1']['b'])
    a1 = _bn_act(y1, st1, p['bn1']['g'], p['bn1']['b'], relu=True)
    y2, st2 = _mm_stats(a1, p['e2']['W'], p['e2']['b'])
    a2 = _bn_act(y2, st2, p['bn2']['g'], p['bn2']['b'], relu=True)
    y3, st3 = _mm_stats(a2, p['e3']['W'], p['e3']['b'])
    a3 = _bn_act(y3, st3, p['bn3']['g'], p['bn3']['b'], relu=True)
    yz1, stz1 = _mm_stats(a3, p['z1']['W'], p['z1']['b'])
    az1 = _bn_act(yz1, stz1, p['bn4']['g'], p['bn4']['b'], relu=False)
    yz2, stz2 = _mm_stats(az1, p['z2']['W'], p['z2']['b'])
    az2 = _bn_act(yz2, stz2, p['bn5']['g'], p['bn5']['b'], relu=False)
    yz3, stz3 = _mm_stats(az2, p['z3']['W'], p['z3']['b'])
    z3 = _bn_act(yz3, stz3, p['bn6']['g'], p['bn6']['b'], relu=False)
    yd1, std1 = _mm_stats(z3, p['d1']['W'], p['d1']['b'])
    ad1 = _bn_act(yd1, std1, p['bn7']['g'], p['bn7']['b'], relu=True)
    yd2, std2 = _mm_stats(ad1, p['d2']['W'], p['d2']['b'])
    ad2 = _bn_act(yd2, std2, p['bn8']['g'], p['bn8']['b'], relu=True)
    yd3, std3 = _mm_stats(ad2, p['d3']['W'], p['d3']['b'])
    ad3 = _bn_act(yd3, std3, p['bn9']['g'], p['bn9']['b'], relu=True)

    x_bar, mean_, disp_, pi_ = _heads(
        ad3, p['xbar']['W'], p['xbar']['b'], p['dm']['W'], p['dm']['b'],
        p['dd']['W'], p['dd']['b'], p['dp']['W'], p['dp']['b'])

    q = _q_assign(z3, p['cluster'].T)

    src = edge_index[0]
    dst = edge_index[1]
    e = src.shape[0]
    nch = -(-e // (CHUNK * NW * NBUF)) * (NW * NBUF)
    pad = nch * CHUNK - e
    srcp = jnp.concatenate([src, jnp.zeros((pad,), jnp.int32)]).reshape(nch, CHUNK)
    wp = jnp.concatenate([edge_weight, jnp.zeros((pad,), jnp.float32)]).reshape(nch, CHUNK)
    dstp = jnp.concatenate([dst, jnp.zeros((pad,), jnp.int32)]).reshape(nch, CHUNK)

    def spmm_all(m_chunks):
        return [_spmm_chunk(m, srcp, dstp, wp) for m in m_chunks]

    m = _mm_chunks(x, p['g1'], [128, 128])
    ps = spmm_all(m)
    m = _gnn_mix(ps, a1, p['g2'], [128, 128])
    ps = spmm_all(m)
    m = _gnn_mix(ps, a2, p['g3'], [128, 128, 128, 128])
    ps = spmm_all(m)
    m = _gnn_mix(ps, a3, p['g4'], [64])
    ps = spmm_all(m)
    m = _gnn_mix(ps, az1, p['g5'], [32])
    ps = spmm_all(m)
    m = _gnn_mix(ps, az2, p['g6'], [16])
    ps = spmm_all(m)
    g7p = jnp.pad(p['g7'], ((0, 0), (0, 6)))
    m = _gnn_mix(ps, z3, g7p, [16])
    ps = spmm_all(m)
    predict = _softmax10(ps[0])

    return (x_bar, q, predict, z3, mean_, disp_, pi_)
